# baseline (speedup 1.0000x reference)
import functools

import jax
import jax.numpy as jnp
from jax import lax
from jax.experimental import pallas as pl
from jax.experimental.pallas import tpu as pltpu
from jax.experimental.pallas import tpu_sc as plsc

D = 128
NC = 2
NS = 16
NW = NC * NS
CB = 128
PAIR = 2
NQ = 2


@functools.lru_cache(maxsize=None)
def _make_gather(S: int, T: int):
    assert S % (NW * CB) == 0 and T % PAIR == 0
    npair = T // PAIR
    assert npair >= 4 and (npair - 3) % NQ == 0

    mesh = plsc.VectorSubcoreMesh(core_axis_name="c", subcore_axis_name="s")

    @functools.partial(
        pl.kernel,
        mesh=mesh,
        out_type=jax.ShapeDtypeStruct((T, S, D), jnp.float32),
        scratch_types=[
            pltpu.VMEM((T, CB), jnp.int32),
            pltpu.VMEM((NQ, PAIR, CB, D), jnp.float32),
        ]
        + [pltpu.SemaphoreType.DMA] * NQ
        + [pltpu.SemaphoreType.DMA] * NQ,
    )
    def gather_kernel(table_hbm, idx_hbm, out_hbm, idx_v, rows_v, *sems):
        gsem = sems[:NQ]
        osem = sems[NQ:]
        wid = lax.axis_index("s") * NC + lax.axis_index("c")
        s0 = wid * CB

        pltpu.sync_copy(idx_hbm.at[:, pl.ds(s0, CB)], idx_v)

        def start_gather(p, q):
            for k in range(PAIR):
                pltpu.async_copy(
                    table_hbm.at[idx_v.at[p * PAIR + k]],
                    rows_v.at[q].at[k],
                    gsem[q],
                )

        def start_out(p, q):
            pltpu.async_copy(
                rows_v.at[q],
                out_hbm.at[pl.ds(p * PAIR, PAIR), pl.ds(s0, CB)],
                osem[q],
            )

        def wait_gather(q):
            pltpu.make_async_copy(
                out_hbm.at[pl.ds(0, PAIR), pl.ds(s0, CB)], rows_v.at[q], gsem[q]
            ).wait()

        def wait_out(q):
            pltpu.make_async_copy(
                rows_v.at[q], out_hbm.at[pl.ds(0, PAIR), pl.ds(s0, CB)], osem[q]
            ).wait()

        start_gather(0, 0)
        start_gather(1, 1)
        wait_gather(0)
        start_out(0, 0)
        wait_out(0)
        start_gather(2, 0)
        wait_gather(1)
        start_out(1, 1)

        def outer(p0, carry):
            for u in range(NQ):
                p = p0 + u
                q = (1 + u) % NQ
                qp = u % NQ
                wait_out(q)
                start_gather(p, q)
                wait_gather(qp)
                start_out(p - 1, qp)
            return carry

        lax.fori_loop(0, (npair - 3) // NQ, lambda i, c: outer(3 + i * NQ, c),
                      0, unroll=False)

        qlast = (npair - 1) % NQ
        wait_gather(qlast)
        start_out(npair - 1, qlast)
        wait_out((npair - 2) % NQ)
        wait_out(qlast)

    return gather_kernel


def kernel(weight, inputs):
    S, T = inputs.shape
    idx_t = jnp.transpose(inputs).astype(jnp.int32)
    out_t = _make_gather(S, T)(weight, idx_t)
    return jnp.transpose(out_t, (1, 0, 2))


# speedup vs baseline: 1.0089x; 1.0089x over previous
import functools

import jax
import jax.numpy as jnp
from jax import lax
from jax.experimental import pallas as pl
from jax.experimental.pallas import tpu as pltpu
from jax.experimental.pallas import tpu_sc as plsc

D = 128
NC = 2
NS = 16
NW = NC * NS
CB = 128
NBUF = 5
LAG = 3


@functools.lru_cache(maxsize=None)
def _make_gather(S: int, T: int):
    assert S % (NW * CB) == 0
    nch = T
    assert nch > NBUF >= LAG + 1 and (nch - NBUF) % NBUF == 0

    mesh = plsc.VectorSubcoreMesh(core_axis_name="c", subcore_axis_name="s")

    @functools.partial(
        pl.kernel,
        mesh=mesh,
        out_type=jax.ShapeDtypeStruct((T, S, D), jnp.float32),
        scratch_types=[
            pltpu.VMEM((T, CB), jnp.int32),
            pltpu.VMEM((NBUF, CB, D), jnp.float32),
        ]
        + [pltpu.SemaphoreType.DMA] * NBUF
        + [pltpu.SemaphoreType.DMA] * NBUF,
    )
    def gather_kernel(table_hbm, idx_hbm, out_hbm, idx_v, rows_v, *sems):
        gsem = sems[:NBUF]
        osem = sems[NBUF:]
        wid = lax.axis_index("s") * NC + lax.axis_index("c")
        s0 = wid * CB

        pltpu.sync_copy(idx_hbm.at[:, pl.ds(s0, CB)], idx_v)

        def start_gather(t, b):
            pltpu.async_copy(table_hbm.at[idx_v.at[t]], rows_v.at[b], gsem[b])

        def start_out(t, b):
            pltpu.async_copy(
                rows_v.at[b], out_hbm.at[t].at[pl.ds(s0, CB)], osem[b]
            )

        def wait_gather(b):
            pltpu.make_async_copy(
                table_hbm.at[pl.ds(0, CB)], rows_v.at[b], gsem[b]
            ).wait()

        def wait_out(b):
            pltpu.make_async_copy(
                rows_v.at[b], out_hbm.at[0].at[pl.ds(s0, CB)], osem[b]
            ).wait()

        for t in range(NBUF):
            start_gather(t, t)
            if t >= LAG:
                wait_gather(t - LAG)
                start_out(t - LAG, t - LAG)

        def outer(j0, carry):
            for u in range(NBUF):
                t = j0 + u
                b = u
                b2 = (u - LAG) % NBUF
                wait_out(b)
                start_gather(t, b)
                wait_gather(b2)
                start_out(t - LAG, b2)
            return carry

        lax.fori_loop(0, (nch - NBUF) // NBUF, lambda i, c: outer(NBUF + i * NBUF, c),
                      0, unroll=False)

        for t in range(nch - LAG, nch):
            b = t % NBUF
            wait_gather(b)
            start_out(t, b)
        for b in range(NBUF):
            wait_out(b)

    return gather_kernel


def kernel(weight, inputs):
    S, T = inputs.shape
    idx_t = jnp.transpose(inputs).astype(jnp.int32)
    out_t = _make_gather(S, T)(weight, idx_t)
    return jnp.transpose(out_t, (1, 0, 2))


# speedup vs baseline: 1.0133x; 1.0044x over previous
import functools

import jax
import jax.numpy as jnp
from jax import lax
from jax.experimental import pallas as pl
from jax.experimental.pallas import tpu as pltpu
from jax.experimental.pallas import tpu_sc as plsc

D = 128
NC = 2
NS = 16
NW = NC * NS
CB = 128
NBUF = 5
LAG = 4


@functools.lru_cache(maxsize=None)
def _make_gather(S: int, T: int):
    assert S % (NW * CB) == 0
    nch = T
    assert nch > NBUF >= LAG + 1 and (nch - NBUF) % NBUF == 0

    mesh = plsc.VectorSubcoreMesh(core_axis_name="c", subcore_axis_name="s")

    @functools.partial(
        pl.kernel,
        mesh=mesh,
        out_type=jax.ShapeDtypeStruct((T, S, D), jnp.float32),
        scratch_types=[
            pltpu.VMEM((T, CB), jnp.int32),
            pltpu.VMEM((NBUF, CB, D), jnp.float32),
        ]
        + [pltpu.SemaphoreType.DMA] * NBUF
        + [pltpu.SemaphoreType.DMA] * NBUF,
    )
    def gather_kernel(table_hbm, idx_hbm, out_hbm, idx_v, rows_v, *sems):
        gsem = sems[:NBUF]
        osem = sems[NBUF:]
        wid = lax.axis_index("s") * NC + lax.axis_index("c")
        s0 = wid * CB

        pltpu.sync_copy(idx_hbm.at[:, pl.ds(s0, CB)], idx_v)

        def start_gather(t, b):
            pltpu.async_copy(table_hbm.at[idx_v.at[t]], rows_v.at[b], gsem[b])

        def start_out(t, b):
            pltpu.async_copy(
                rows_v.at[b], out_hbm.at[t].at[pl.ds(s0, CB)], osem[b]
            )

        def wait_gather(b):
            pltpu.make_async_copy(
                table_hbm.at[pl.ds(0, CB)], rows_v.at[b], gsem[b]
            ).wait()

        def wait_out(b):
            pltpu.make_async_copy(
                rows_v.at[b], out_hbm.at[0].at[pl.ds(s0, CB)], osem[b]
            ).wait()

        for t in range(NBUF):
            start_gather(t, t)
            if t >= LAG:
                wait_gather(t - LAG)
                start_out(t - LAG, t - LAG)

        def outer(j0, carry):
            for u in range(NBUF):
                t = j0 + u
                b = u
                b2 = (u - LAG) % NBUF
                wait_out(b)
                start_gather(t, b)
                wait_gather(b2)
                start_out(t - LAG, b2)
            return carry

        lax.fori_loop(0, (nch - NBUF) // NBUF, lambda i, c: outer(NBUF + i * NBUF, c),
                      0, unroll=False)

        for t in range(nch - LAG, nch):
            b = t % NBUF
            wait_gather(b)
            start_out(t, b)
        for b in range(NBUF):
            wait_out(b)

    return gather_kernel


def kernel(weight, inputs):
    S, T = inputs.shape
    idx_t = jnp.transpose(inputs).astype(jnp.int32)
    out_t = _make_gather(S, T)(weight, idx_t)
    return jnp.transpose(out_t, (1, 0, 2))
